# baseline (speedup 1.0000x reference)
import jax
import jax.numpy as jnp
from jax import lax
from jax.experimental import pallas as pl
from jax.experimental.pallas import tpu as pltpu
from jax.experimental.pallas import tpu_sc as plsc

N_ROWS = 100000
D = 512
N_SEG = 1024

NC = 2
NS = 16
NW = NC * NS
LANES = 16

SEG_PER_W = N_SEG // NW
CHUNK = 64
NSLOT = 3
NU = D // LANES


def _sc_body(x_hbm, idx_hbm, b_hbm, out_hbm,
             xbufs, ibufs, b_v, acc, xsems, isems):
    c = lax.axis_index("c")
    s = lax.axis_index("s")
    w = c * NS + s
    g0 = w * SEG_PER_W

    pltpu.sync_copy(b_hbm, b_v)
    rs = b_v[pl.ds(g0, LANES)][0]
    re = b_v[pl.ds(g0 + SEG_PER_W, LANES)][0]

    zero = jnp.zeros((LANES,), jnp.float32)

    def zbody(t, carry):
        for u in range(NU):
            acc[t, pl.ds(u * LANES, LANES)] = zero
        return carry

    lax.fori_loop(0, SEG_PER_W, zbody, jnp.int32(0))

    cs0 = pl.multiple_of((rs // 8) * 8, 8)
    nch = (re - cs0 + CHUNK - 1) // CHUNK

    def chunk_start(k):
        return pl.multiple_of(
            jnp.minimum(cs0 + k * CHUNK, N_ROWS - CHUNK), 8)

    def issue(k, p):
        cs = chunk_start(k)
        pltpu.async_copy(x_hbm.at[pl.ds(cs, CHUNK)], xbufs[p], xsems[p])
        pltpu.async_copy(idx_hbm.at[pl.ds(cs, CHUNK)],
                         ibufs[p].at[pl.ds(0, CHUNK)], isems[p])

    for p in range(NSLOT):
        @pl.when(p < nch)
        def _():
            issue(jnp.int32(p), p)

    def process(k, p):
        cs = chunk_start(k)
        jlo = jnp.maximum(rs, cs0 + k * CHUNK) - cs
        jhi = jnp.minimum(re, cs + CHUNK) - cs

        tlo = ibufs[p][pl.ds(jlo, LANES)][0]
        thi = ibufs[p][pl.ds(jnp.maximum(jhi - 1, jlo), LANES)][0]
        npieces = jnp.where(jhi > jlo, thi - tlo + 1, 0)

        def piece(i, j):
            done = j >= jhi
            jj = jnp.minimum(j, CHUNK - 1)
            t_raw = ibufs[p][pl.ds(jj, LANES)][0] - g0
            t = jnp.clip(t_raw, 0, SEG_PER_W - 1)
            e = b_v[pl.ds(g0 + t + 1, LANES)][0] - cs
            jend = jnp.where(done, j, jnp.minimum(e, jhi))
            av = tuple(acc[t, pl.ds(u * LANES, LANES)] for u in range(NU))

            def row(r, av):
                return tuple(
                    av[u] + xbufs[p][r, pl.ds(u * LANES, LANES)]
                    for u in range(NU))

            av = lax.fori_loop(j, jend, row, av)
            for u in range(NU):
                acc[t, pl.ds(u * LANES, LANES)] = av[u]
            return jend

        lax.fori_loop(0, npieces, piece, jlo)

    def body(kk, carry):
        for p in range(NSLOT):
            k = kk * NSLOT + p

            @pl.when(k < nch)
            def _():
                pltpu.make_async_copy(
                    x_hbm.at[pl.ds(0, CHUNK)], xbufs[p], xsems[p]).wait()
                pltpu.make_async_copy(
                    idx_hbm.at[pl.ds(0, CHUNK)],
                    ibufs[p].at[pl.ds(0, CHUNK)], isems[p]).wait()

                process(k, p)

                @pl.when(k + NSLOT < nch)
                def _():
                    issue(k + NSLOT, p)
        return carry

    lax.fori_loop(0, (nch + NSLOT - 1) // NSLOT, body, jnp.int32(0))

    pltpu.sync_copy(acc, out_hbm.at[pl.ds(g0, SEG_PER_W)])


def kernel(X, graph_idx, n):
    num_segments = n.shape[0]
    idx32 = graph_idx.astype(jnp.int32)
    edges = jnp.arange(num_segments + 1, dtype=jnp.int32)
    bounds = jnp.searchsorted(idx32, edges, side="left").astype(jnp.int32)
    bounds = jnp.concatenate(
        [bounds, jnp.full((2 * LANES - 1,), N_ROWS, jnp.int32)])

    sc = pl.kernel(
        _sc_body,
        out_type=jax.ShapeDtypeStruct((N_SEG, D), jnp.float32),
        mesh=plsc.VectorSubcoreMesh(core_axis_name="c", subcore_axis_name="s"),
        scratch_types=[
            [pltpu.VMEM((CHUNK, D), jnp.float32) for _ in range(NSLOT)],
            [pltpu.VMEM((CHUNK + LANES,), jnp.int32) for _ in range(NSLOT)],
            pltpu.VMEM((N_SEG + 2 * LANES,), jnp.int32),
            pltpu.VMEM((SEG_PER_W, D), jnp.float32),
            [pltpu.SemaphoreType.DMA for _ in range(NSLOT)],
            [pltpu.SemaphoreType.DMA for _ in range(NSLOT)],
        ],
    )
    return sc(X, idx32, bounds)


# speedup vs baseline: 1.0045x; 1.0045x over previous
import jax
import jax.numpy as jnp
from jax import lax
from jax.experimental import pallas as pl
from jax.experimental.pallas import tpu as pltpu
from jax.experimental.pallas import tpu_sc as plsc

N_ROWS = 100000
D = 512
N_SEG = 1024

NC = 2
NS = 16
NW = NC * NS
LANES = 16

SEG_PER_W = N_SEG // NW
CHUNK = 64
NSLOT = 3
NU = D // LANES


def _sc_body(x_hbm, idx_hbm, b_hbm, out_hbm,
             xbufs, ibufs, b_v, acc, xsems, isems):
    c = lax.axis_index("c")
    s = lax.axis_index("s")
    w = c * NS + s
    g0 = w * SEG_PER_W

    pltpu.sync_copy(b_hbm, b_v)
    rs = b_v[pl.ds(g0, LANES)][0]
    re = b_v[pl.ds(g0 + SEG_PER_W, LANES)][0]

    zero = jnp.zeros((LANES,), jnp.float32)

    def zbody(t, carry):
        for u in range(NU):
            acc[t, pl.ds(u * LANES, LANES)] = zero
        return carry

    lax.fori_loop(0, SEG_PER_W, zbody, jnp.int32(0))

    cs0 = pl.multiple_of((rs // 8) * 8, 8)
    nch = (re - cs0 + CHUNK - 1) // CHUNK

    def chunk_start(k):
        return pl.multiple_of(
            jnp.minimum(cs0 + k * CHUNK, N_ROWS - CHUNK), 8)

    def issue(k, p):
        cs = chunk_start(k)
        pltpu.async_copy(x_hbm.at[pl.ds(cs, CHUNK)], xbufs[p], xsems[p])
        pltpu.async_copy(idx_hbm.at[pl.ds(cs, CHUNK)],
                         ibufs[p].at[pl.ds(0, CHUNK)], isems[p])

    for p in range(NSLOT):
        @pl.when(p < nch)
        def _():
            issue(jnp.int32(p), p)

    def process(k, p):
        cs = chunk_start(k)
        jlo = jnp.maximum(rs, cs0 + k * CHUNK) - cs
        jhi = jnp.minimum(re, cs + CHUNK) - cs

        tlo = ibufs[p][pl.ds(jlo, LANES)][0]
        thi = ibufs[p][pl.ds(jnp.maximum(jhi - 1, jlo), LANES)][0]
        npieces = jnp.where(jhi > jlo, thi - tlo + 1, 0)

        def piece(i, j):
            done = j >= jhi
            jj = jnp.minimum(j, CHUNK - 1)
            t_raw = ibufs[p][pl.ds(jj, LANES)][0] - g0
            t = jnp.clip(t_raw, 0, SEG_PER_W - 1)
            e = b_v[pl.ds(g0 + t + 1, LANES)][0] - cs
            jend = jnp.where(done, j, jnp.minimum(e, jhi))
            av = tuple(acc[t, pl.ds(u * LANES, LANES)] for u in range(NU))

            def row(r, av):
                return tuple(
                    av[u] + xbufs[p][r, pl.ds(u * LANES, LANES)]
                    for u in range(NU))

            av = lax.fori_loop(j, jend, row, av)
            for u in range(NU):
                acc[t, pl.ds(u * LANES, LANES)] = av[u]
            return jend

        lax.fori_loop(0, npieces, piece, jlo)

    def body(kk, carry):
        for p in range(NSLOT):
            k = kk * NSLOT + p

            @pl.when(k < nch)
            def _():
                pltpu.make_async_copy(
                    x_hbm.at[pl.ds(0, CHUNK)], xbufs[p], xsems[p]).wait()
                pltpu.make_async_copy(
                    idx_hbm.at[pl.ds(0, CHUNK)],
                    ibufs[p].at[pl.ds(0, CHUNK)], isems[p]).wait()

                process(k, p)

                @pl.when(k + NSLOT < nch)
                def _():
                    issue(k + NSLOT, p)
        return carry

    lax.fori_loop(0, (nch + NSLOT - 1) // NSLOT, body, jnp.int32(0))

    pltpu.sync_copy(acc, out_hbm.at[pl.ds(g0, SEG_PER_W)])


def kernel(X, graph_idx, n):
    num_segments = n.shape[0]
    idx32 = graph_idx.astype(jnp.int32)
    edges = jnp.arange(num_segments + 1, dtype=jnp.int32)
    bounds = jnp.searchsorted(
        idx32, edges, side="left", method="scan_unrolled").astype(jnp.int32)
    bounds = jnp.concatenate(
        [bounds, jnp.full((2 * LANES - 1,), N_ROWS, jnp.int32)])

    sc = pl.kernel(
        _sc_body,
        out_type=jax.ShapeDtypeStruct((N_SEG, D), jnp.float32),
        mesh=plsc.VectorSubcoreMesh(core_axis_name="c", subcore_axis_name="s"),
        scratch_types=[
            [pltpu.VMEM((CHUNK, D), jnp.float32) for _ in range(NSLOT)],
            [pltpu.VMEM((CHUNK + LANES,), jnp.int32) for _ in range(NSLOT)],
            pltpu.VMEM((N_SEG + 2 * LANES,), jnp.int32),
            pltpu.VMEM((SEG_PER_W, D), jnp.float32),
            [pltpu.SemaphoreType.DMA for _ in range(NSLOT)],
            [pltpu.SemaphoreType.DMA for _ in range(NSLOT)],
        ],
    )
    return sc(X, idx32, bounds)


# speedup vs baseline: 1.4165x; 1.4101x over previous
import jax
import jax.numpy as jnp
from jax import lax
from jax.experimental import pallas as pl
from jax.experimental.pallas import tpu as pltpu
from jax.experimental.pallas import tpu_sc as plsc

N_ROWS = 100000
D = 512
N_SEG = 1024

NC = 2
NS = 16
NW = NC * NS
LANES = 16

SEG_PER_W = N_SEG // NW
CHUNK = 64
NSLOT = 3
NU = D // LANES


def _sc_body(x_hbm, idx_hbm, b_hbm, out_hbm,
             xbufs, ibufs, b_v, acc, xsems, isems):
    c = lax.axis_index("c")
    s = lax.axis_index("s")
    w = c * NS + s
    g0 = w * SEG_PER_W

    pltpu.sync_copy(b_hbm, b_v)
    rs = b_v[pl.ds(g0, LANES)][0]
    re = b_v[pl.ds(g0 + SEG_PER_W, LANES)][0]

    zero = jnp.zeros((LANES,), jnp.float32)

    def zbody(t, carry):
        for u in range(NU):
            acc[t, pl.ds(u * LANES, LANES)] = zero
        return carry

    lax.fori_loop(0, SEG_PER_W, zbody, jnp.int32(0))

    cs0 = pl.multiple_of((rs // 8) * 8, 8)
    nch = (re - cs0 + CHUNK - 1) // CHUNK

    def chunk_start(k):
        return pl.multiple_of(
            jnp.minimum(cs0 + k * CHUNK, N_ROWS - CHUNK), 8)

    def issue(k, p):
        cs = chunk_start(k)
        pltpu.async_copy(x_hbm.at[pl.ds(cs, CHUNK)], xbufs[p], xsems[p])
        pltpu.async_copy(idx_hbm.at[pl.ds(cs, CHUNK)],
                         ibufs[p].at[pl.ds(0, CHUNK)], isems[p])

    for p in range(NSLOT):
        @pl.when(p < nch)
        def _():
            issue(jnp.int32(p), p)

    def process(k, p):
        cs = chunk_start(k)
        jlo = jnp.maximum(rs, cs0 + k * CHUNK) - cs
        jhi = jnp.minimum(re, cs + CHUNK) - cs

        tlo = ibufs[p][pl.ds(jlo, LANES)][0]
        thi = ibufs[p][pl.ds(jnp.maximum(jhi - 1, jlo), LANES)][0]
        npieces = jnp.where(jhi > jlo, thi - tlo + 1, 0)

        def piece(i, j):
            done = j >= jhi
            jj = jnp.minimum(j, CHUNK - 1)
            t_raw = ibufs[p][pl.ds(jj, LANES)][0] - g0
            t = jnp.clip(t_raw, 0, SEG_PER_W - 1)
            e = b_v[pl.ds(g0 + t + 1, LANES)][0] - cs
            jend = jnp.where(done, j, jnp.minimum(e, jhi))
            av = tuple(acc[t, pl.ds(u * LANES, LANES)] for u in range(NU))

            def row(r, av):
                return tuple(
                    av[u] + xbufs[p][r, pl.ds(u * LANES, LANES)]
                    for u in range(NU))

            av = lax.fori_loop(j, jend, row, av)
            for u in range(NU):
                acc[t, pl.ds(u * LANES, LANES)] = av[u]
            return jend

        lax.fori_loop(0, npieces, piece, jlo)

    def body(kk, carry):
        for p in range(NSLOT):
            k = kk * NSLOT + p

            @pl.when(k < nch)
            def _():
                pltpu.make_async_copy(
                    x_hbm.at[pl.ds(0, CHUNK)], xbufs[p], xsems[p]).wait()
                pltpu.make_async_copy(
                    idx_hbm.at[pl.ds(0, CHUNK)],
                    ibufs[p].at[pl.ds(0, CHUNK)], isems[p]).wait()

                process(k, p)

                @pl.when(k + NSLOT < nch)
                def _():
                    issue(k + NSLOT, p)
        return carry

    lax.fori_loop(0, (nch + NSLOT - 1) // NSLOT, body, jnp.int32(0))

    pltpu.sync_copy(acc, out_hbm.at[pl.ds(g0, SEG_PER_W)])


def kernel(X, graph_idx, n):
    num_segments = n.shape[0]
    idx32 = graph_idx.astype(jnp.int32)
    edges = jnp.arange(num_segments + 1, dtype=jnp.int32)
    bounds = jnp.sum(
        (idx32[None, :] < edges[:, None]).astype(jnp.int32),
        axis=1, dtype=jnp.int32)
    bounds = jnp.concatenate(
        [bounds, jnp.full((2 * LANES - 1,), N_ROWS, jnp.int32)])

    sc = pl.kernel(
        _sc_body,
        out_type=jax.ShapeDtypeStruct((N_SEG, D), jnp.float32),
        mesh=plsc.VectorSubcoreMesh(core_axis_name="c", subcore_axis_name="s"),
        scratch_types=[
            [pltpu.VMEM((CHUNK, D), jnp.float32) for _ in range(NSLOT)],
            [pltpu.VMEM((CHUNK + LANES,), jnp.int32) for _ in range(NSLOT)],
            pltpu.VMEM((N_SEG + 2 * LANES,), jnp.int32),
            pltpu.VMEM((SEG_PER_W, D), jnp.float32),
            [pltpu.SemaphoreType.DMA for _ in range(NSLOT)],
            [pltpu.SemaphoreType.DMA for _ in range(NSLOT)],
        ],
    )
    return sc(X, idx32, bounds)


# speedup vs baseline: 1.4166x; 1.0000x over previous
import jax
import jax.numpy as jnp
from jax import lax
from jax.experimental import pallas as pl
from jax.experimental.pallas import tpu as pltpu
from jax.experimental.pallas import tpu_sc as plsc

N_ROWS = 100000
D = 512
N_SEG = 1024

NC = 2
NS = 16
NW = NC * NS
LANES = 16

SEG_PER_W = N_SEG // NW
CHUNK = 64
NSLOT = 3
NU = D // LANES


def _sc_body(x_hbm, idx_hbm, b_hbm, out_hbm,
             xbufs, ibufs, b_v, acc, xsems, isems):
    c = lax.axis_index("c")
    s = lax.axis_index("s")
    w = c * NS + s
    g0 = w * SEG_PER_W

    pltpu.sync_copy(b_hbm, b_v)
    rs = b_v[pl.ds(g0, LANES)][0]
    re = b_v[pl.ds(g0 + SEG_PER_W, LANES)][0]

    zero = jnp.zeros((LANES,), jnp.float32)

    def zbody(t, carry):
        for u in range(NU):
            acc[t, pl.ds(u * LANES, LANES)] = zero
        return carry

    lax.fori_loop(0, SEG_PER_W, zbody, jnp.int32(0))

    cs0 = pl.multiple_of((rs // 8) * 8, 8)
    nch = (re - cs0 + CHUNK - 1) // CHUNK

    def chunk_start(k):
        return pl.multiple_of(
            jnp.minimum(cs0 + k * CHUNK, N_ROWS - CHUNK), 8)

    def issue(k, p):
        cs = chunk_start(k)
        pltpu.async_copy(x_hbm.at[pl.ds(cs, CHUNK)], xbufs[p], xsems[p])
        pltpu.async_copy(idx_hbm.at[pl.ds(cs, CHUNK)],
                         ibufs[p].at[pl.ds(0, CHUNK)], isems[p])

    for p in range(NSLOT):
        @pl.when(p < nch)
        def _():
            issue(jnp.int32(p), p)

    def process(k, p):
        cs = chunk_start(k)
        jlo = jnp.maximum(rs, cs0 + k * CHUNK) - cs
        jhi = jnp.minimum(re, cs + CHUNK) - cs

        tlo = ibufs[p][pl.ds(jlo, LANES)][0]
        thi = ibufs[p][pl.ds(jnp.maximum(jhi - 1, jlo), LANES)][0]
        npieces = jnp.where(jhi > jlo, thi - tlo + 1, 0)

        def piece(i, j):
            done = j >= jhi
            jj = jnp.minimum(j, CHUNK - 1)
            t_raw = ibufs[p][pl.ds(jj, LANES)][0] - g0
            t = jnp.clip(t_raw, 0, SEG_PER_W - 1)
            e = b_v[pl.ds(g0 + t + 1, LANES)][0] - cs
            jend = jnp.where(done, j, jnp.minimum(e, jhi))
            av = tuple(acc[t, pl.ds(u * LANES, LANES)] for u in range(NU))

            def row(r, av):
                return tuple(
                    av[u] + xbufs[p][r, pl.ds(u * LANES, LANES)]
                    for u in range(NU))

            av = lax.fori_loop(j, jend, row, av)
            for u in range(NU):
                acc[t, pl.ds(u * LANES, LANES)] = av[u]
            return jend

        lax.fori_loop(0, npieces, piece, jlo)

    def body(kk, carry):
        for p in range(NSLOT):
            k = kk * NSLOT + p

            @pl.when(k < nch)
            def _():
                pltpu.make_async_copy(
                    x_hbm.at[pl.ds(0, CHUNK)], xbufs[p], xsems[p]).wait()
                pltpu.make_async_copy(
                    idx_hbm.at[pl.ds(0, CHUNK)],
                    ibufs[p].at[pl.ds(0, CHUNK)], isems[p]).wait()

                process(k, p)

                @pl.when(k + NSLOT < nch)
                def _():
                    issue(k + NSLOT, p)
        return carry

    lax.fori_loop(0, (nch + NSLOT - 1) // NSLOT, body, jnp.int32(0))

    pltpu.sync_copy(acc, out_hbm.at[pl.ds(g0, SEG_PER_W)])


def kernel(X, graph_idx, n):
    num_segments = n.shape[0]
    idx32 = graph_idx.astype(jnp.int32)
    edges = jnp.arange(num_segments + 1, dtype=jnp.int32)
    bounds = jnp.sum(
        (idx32[:, None] < edges[None, :]).astype(jnp.int32),
        axis=0, dtype=jnp.int32)
    bounds = jnp.concatenate(
        [bounds, jnp.full((2 * LANES - 1,), N_ROWS, jnp.int32)])

    sc = pl.kernel(
        _sc_body,
        out_type=jax.ShapeDtypeStruct((N_SEG, D), jnp.float32),
        mesh=plsc.VectorSubcoreMesh(core_axis_name="c", subcore_axis_name="s"),
        scratch_types=[
            [pltpu.VMEM((CHUNK, D), jnp.float32) for _ in range(NSLOT)],
            [pltpu.VMEM((CHUNK + LANES,), jnp.int32) for _ in range(NSLOT)],
            pltpu.VMEM((N_SEG + 2 * LANES,), jnp.int32),
            pltpu.VMEM((SEG_PER_W, D), jnp.float32),
            [pltpu.SemaphoreType.DMA for _ in range(NSLOT)],
            [pltpu.SemaphoreType.DMA for _ in range(NSLOT)],
        ],
    )
    return sc(X, idx32, bounds)
